# baseline (speedup 1.0000x reference)
import functools

import jax
import jax.numpy as jnp
from jax import lax
from jax.experimental import pallas as pl
from jax.experimental.pallas import tpu as pltpu
from jax.experimental.pallas import tpu_sc as plsc

B = 64
S = 512
H = 768
EPS = 1e-12
NC = 2
NS = 16
NW = NC * NS
SPT = S // NW
HV = H // 16

_GATHER_DNUMS = lax.GatherDimensionNumbers(
    offset_dims=(), collapsed_slice_dims=(0,), start_index_map=(0,))


def _lane_sum(x):
    lanes = lax.iota(jnp.int32, 16)
    for sh in (8, 4, 2, 1):
        idx = (lanes + sh) & 15
        x = x + lax.gather(x, idx[:, None], _GATHER_DNUMS, (1,),
                           mode=lax.GatherScatterMode.PROMISE_IN_BOUNDS)
    return x


def _rsqrt_vec(x):
    half = jnp.full((16,), 0.5, jnp.float32)
    three_half = jnp.full((16,), 1.5, jnp.float32)
    i = plsc.bitcast(x, jnp.int32)
    i = jnp.full((16,), 0x5F3759DF, jnp.int32) - lax.shift_right_arithmetic(i, 1)
    y = plsc.bitcast(i, jnp.float32)
    hx = half * x
    for _ in range(2):
        y = y * (three_half - hx * y * y)
    return y


def _tile_body(ids_hbm, word_hbm, pos_hbm, tok_hbm, g_hbm, bt_hbm, out_hbm,
               idx_v, bias_v, tok_v, gamma_v, beta_v, buf0, buf1, sem0, sem1):
    c = lax.axis_index("c")
    s_ = lax.axis_index("s")
    w = s_ * NC + c

    pltpu.sync_copy(ids_hbm.at[w], idx_v)
    pltpu.sync_copy(pos_hbm.at[pl.ds(w * SPT, SPT)], bias_v)
    pltpu.sync_copy(tok_hbm.at[0], tok_v)
    pltpu.sync_copy(g_hbm, gamma_v)
    pltpu.sync_copy(bt_hbm, beta_v)

    def _fold(sl, carry):
        for j in range(HV):
            d = pl.ds(j * 16, 16)
            bias_v[sl, d] = bias_v[sl, d] + tok_v[d]
        return carry
    lax.fori_loop(0, SPT, _fold, 0)

    one_over_h = jnp.full((16,), 1.0 / H, jnp.float32)
    eps_v = jnp.full((16,), EPS, jnp.float32)

    KEEP = 24

    def _one_row(buf, r):
        sl = r & (SPT - 1)
        s0 = s1 = q0 = q1 = jnp.zeros((16,), jnp.float32)
        xs = []
        for j in range(HV):
            d = pl.ds(j * 16, 16)
            xb = buf[r, d] + bias_v[sl, d]
            if j < KEEP:
                xs.append(xb)
            else:
                buf[r, d] = xb
            if j & 1:
                s1 = s1 + xb
                q1 = q1 + xb * xb
            else:
                s0 = s0 + xb
                q0 = q0 + xb * xb
        mean = _lane_sum(s0 + s1) * one_over_h
        var = _lane_sum(q0 + q1) * one_over_h - mean * mean
        istd = _rsqrt_vec(var + eps_v)
        for j in range(HV):
            d = pl.ds(j * 16, 16)
            xb = xs[j] if j < KEEP else buf[r, d]
            t = (xb - mean) * istd
            buf[r, d] = t * gamma_v[d] + beta_v[d]

    CB = 2 * SPT

    def _ln_chunk(buf):
        def _r_step(r, carry):
            _one_row(buf, r)
            return carry
        lax.fori_loop(0, CB, _r_step, 0)

    def _start_gather(cb, buf, sem):
        pltpu.async_copy(word_hbm.at[idx_v.at[pl.ds(cb * CB, CB)]], buf, sem)

    def _wait_gather(cb, buf, sem):
        pltpu.make_async_copy(
            word_hbm.at[idx_v.at[pl.ds(cb * CB, CB)]], buf, sem).wait()

    def _finish(cb, buf):
        _ln_chunk(buf)
        b0 = cb * 2
        pltpu.sync_copy(buf.at[pl.ds(0, SPT)],
                        out_hbm.at[pl.ds(b0 * S + w * SPT, SPT)])
        pltpu.sync_copy(buf.at[pl.ds(SPT, SPT)],
                        out_hbm.at[pl.ds((b0 + 1) * S + w * SPT, SPT)])

    NCHUNK = B // 2
    _start_gather(0, buf0, sem0)

    def _b_step(i, carry):
        c0 = i * 2
        _start_gather(c0 + 1, buf1, sem1)
        _wait_gather(c0, buf0, sem0)
        _finish(c0, buf0)

        @pl.when(i < NCHUNK // 2 - 1)
        def _():
            _start_gather(c0 + 2, buf0, sem0)
        _wait_gather(c0 + 1, buf1, sem1)
        _finish(c0 + 1, buf1)
        return carry
    lax.fori_loop(0, NCHUNK // 2, _b_step, 0)


_sc_call = functools.partial(
    pl.kernel,
    out_type=jax.ShapeDtypeStruct((B * S, H), jnp.float32),
    mesh=plsc.VectorSubcoreMesh(core_axis_name="c", subcore_axis_name="s"),
    compiler_params=pltpu.CompilerParams(needs_layout_passes=False),
    scratch_types=[
        pltpu.VMEM((B * SPT,), jnp.int32),
        pltpu.VMEM((SPT, H), jnp.float32),
        pltpu.VMEM((H,), jnp.float32),
        pltpu.VMEM((H,), jnp.float32),
        pltpu.VMEM((H,), jnp.float32),
        pltpu.VMEM((2 * SPT, H), jnp.float32),
        pltpu.VMEM((2 * SPT, H), jnp.float32),
        pltpu.SemaphoreType.DMA,
        pltpu.SemaphoreType.DMA,
    ],
)(_tile_body)


def kernel(input_ids, word_emb, pos_emb, tok_emb, gamma, beta):
    ids = input_ids.astype(jnp.int32).reshape(B, NW, SPT)
    ids = ids.transpose(1, 0, 2).reshape(NW, B * SPT)
    out = _sc_call(ids, word_emb, pos_emb, tok_emb, gamma, beta)
    return out.reshape(B, S, H)


# speedup vs baseline: 1.2347x; 1.2347x over previous
import functools

import jax
import jax.numpy as jnp
from jax import lax
from jax.experimental import pallas as pl
from jax.experimental.pallas import tpu as pltpu
from jax.experimental.pallas import tpu_sc as plsc

B = 64
S = 512
H = 768
EPS = 1e-12
NC = 2
NS = 16
NW = NC * NS
SPT = S // NW
HV = H // 16

_GATHER_DNUMS = lax.GatherDimensionNumbers(
    offset_dims=(), collapsed_slice_dims=(0,), start_index_map=(0,))


def _lane_sum(x):
    lanes = lax.iota(jnp.int32, 16)
    for sh in (8, 4, 2, 1):
        idx = (lanes + sh) & 15
        x = x + lax.gather(x, idx[:, None], _GATHER_DNUMS, (1,),
                           mode=lax.GatherScatterMode.PROMISE_IN_BOUNDS)
    return x


def _rsqrt_vec(x):
    half = jnp.full((16,), 0.5, jnp.float32)
    three_half = jnp.full((16,), 1.5, jnp.float32)
    i = plsc.bitcast(x, jnp.int32)
    i = jnp.full((16,), 0x5F3759DF, jnp.int32) - lax.shift_right_arithmetic(i, 1)
    y = plsc.bitcast(i, jnp.float32)
    hx = half * x
    for _ in range(2):
        y = y * (three_half - hx * y * y)
    return y


def _tile_body(ids_hbm, word_hbm, pos_hbm, tok_hbm, g_hbm, bt_hbm, out_hbm,
               idx_v, bias_v, tok_v, gamma_v, beta_v,
               in0, in1, ob0, ob1, sg0, sg1, so0, so1):
    c = lax.axis_index("c")
    s_ = lax.axis_index("s")
    w = s_ * NC + c

    pltpu.sync_copy(ids_hbm.at[w], idx_v)
    pltpu.sync_copy(pos_hbm.at[pl.ds(w * SPT, SPT)], bias_v)
    pltpu.sync_copy(tok_hbm.at[0], tok_v)
    pltpu.sync_copy(g_hbm, gamma_v)
    pltpu.sync_copy(bt_hbm, beta_v)

    def _fold(sl, carry):
        for j in range(HV):
            d = pl.ds(j * 16, 16)
            bias_v[sl, d] = bias_v[sl, d] + tok_v[d]
        return carry
    lax.fori_loop(0, SPT, _fold, 0)

    one_over_h = jnp.full((16,), 1.0 / H, jnp.float32)
    eps_v = jnp.full((16,), EPS, jnp.float32)

    KEEP = 24

    def _one_row(src, dst, r):
        s0 = s1 = q0 = q1 = jnp.zeros((16,), jnp.float32)
        xs = []
        for j in range(HV):
            d = pl.ds(j * 16, 16)
            xb = src[r, d] + bias_v[r, d]
            if j < KEEP:
                xs.append(xb)
            else:
                dst[r, d] = xb
            if j & 1:
                s1 = s1 + xb
                q1 = q1 + xb * xb
            else:
                s0 = s0 + xb
                q0 = q0 + xb * xb
        mean = _lane_sum(s0 + s1) * one_over_h
        var = _lane_sum(q0 + q1) * one_over_h - mean * mean
        istd = _rsqrt_vec(var + eps_v)
        for j in range(HV):
            d = pl.ds(j * 16, 16)
            xb = xs[j] if j < KEEP else dst[r, d]
            t = (xb - mean) * istd
            dst[r, d] = t * gamma_v[d] + beta_v[d]

    def _ln16(src, dst):
        def _r_step(r, carry):
            _one_row(src, dst, r)
            return carry
        lax.fori_loop(0, SPT, _r_step, 0)

    def _start_gather(b, buf, sem):
        pltpu.async_copy(word_hbm.at[idx_v.at[pl.ds(b * SPT, SPT)]], buf, sem)

    def _wait_gather(b, buf, sem):
        pltpu.make_async_copy(
            word_hbm.at[idx_v.at[pl.ds(b * SPT, SPT)]], buf, sem).wait()

    def _out_slice(b):
        return out_hbm.at[pl.ds(b * S + w * SPT, SPT)]

    _start_gather(0, in0, sg0)
    _start_gather(1, in1, sg1)

    def _b_step(i, carry):
        for p, (inb, ob, sg, so) in enumerate(
                ((in0, ob0, sg0, so0), (in1, ob1, sg1, so1))):
            b = i * 2 + p
            _wait_gather(b, inb, sg)

            @pl.when(i > 0)
            def _():
                pltpu.make_async_copy(ob, _out_slice(b - 2), so).wait()
            _ln16(inb, ob)
            pltpu.async_copy(ob, _out_slice(b), so)

            @pl.when(b + 2 < B)
            def _():
                _start_gather(b + 2, inb, sg)
        return carry
    lax.fori_loop(0, B // 2, _b_step, 0)

    pltpu.make_async_copy(ob0, _out_slice(B - 2), so0).wait()
    pltpu.make_async_copy(ob1, _out_slice(B - 1), so1).wait()


_sc_call = functools.partial(
    pl.kernel,
    out_type=jax.ShapeDtypeStruct((B * S, H), jnp.float32),
    mesh=plsc.VectorSubcoreMesh(core_axis_name="c", subcore_axis_name="s"),
    compiler_params=pltpu.CompilerParams(needs_layout_passes=False),
    scratch_types=[
        pltpu.VMEM((B * SPT,), jnp.int32),
        pltpu.VMEM((SPT, H), jnp.float32),
        pltpu.VMEM((H,), jnp.float32),
        pltpu.VMEM((H,), jnp.float32),
        pltpu.VMEM((H,), jnp.float32),
        pltpu.VMEM((SPT, H), jnp.float32),
        pltpu.VMEM((SPT, H), jnp.float32),
        pltpu.VMEM((SPT, H), jnp.float32),
        pltpu.VMEM((SPT, H), jnp.float32),
        pltpu.SemaphoreType.DMA,
        pltpu.SemaphoreType.DMA,
        pltpu.SemaphoreType.DMA,
        pltpu.SemaphoreType.DMA,
    ],
)(_tile_body)


def kernel(input_ids, word_emb, pos_emb, tok_emb, gamma, beta):
    ids = input_ids.astype(jnp.int32).reshape(B, NW, SPT)
    ids = ids.transpose(1, 0, 2).reshape(NW, B * SPT)
    out = _sc_call(ids, word_emb, pos_emb, tok_emb, gamma, beta)
    return out.reshape(B, S, H)
